# Initial kernel scaffold; baseline (speedup 1.0000x reference)
import jax
import jax.numpy as jnp
from jax.experimental import pallas as pl


def kernel(data, params):
    raise NotImplementedError("write your pallas kernel here")


# speedup vs baseline: 4.2617x; 4.2617x over previous
import functools

import numpy as np
import jax
import jax.numpy as jnp
from jax import lax
from jax.experimental import pallas as pl
from jax.experimental.pallas import tpu as pltpu

_B = 16
_N = 1024
_K = 64
_EPS = 1e-5
_NQ1 = 512
_NQ2 = 128
_INF = float('inf')


def _fps_body(px_ref, py_ref, pz_ref,
              q1x_ref, q1y_ref, q1z_ref, q2x_ref, q2y_ref, q2z_ref):
    def run(px, py, pz, n_out):
        b, n = px.shape
        lane = lax.broadcasted_iota(jnp.int32, (b, n), 1)
        oio = lax.broadcasted_iota(jnp.int32, (b, n_out), 1)

        def body(i, c):
            qx, qy, qz, dists, far = c
            selm = lane == far
            cx = jnp.sum(jnp.where(selm, px, 0.0), axis=1, keepdims=True)
            cy = jnp.sum(jnp.where(selm, py, 0.0), axis=1, keepdims=True)
            cz = jnp.sum(jnp.where(selm, pz, 0.0), axis=1, keepdims=True)
            qx = jnp.where(oio == i, cx, qx)
            qy = jnp.where(oio == i, cy, qy)
            qz = jnp.where(oio == i, cz, qz)
            dx = px - cx
            dy = py - cy
            dz = pz - cz
            d = (dx * dx + dy * dy) + dz * dz
            dists = jnp.minimum(dists, d)
            m = jnp.max(dists, axis=1, keepdims=True)
            far = jnp.min(jnp.where(dists == m, lane, n), axis=1,
                          keepdims=True)
            return qx, qy, qz, dists, far

        z = jnp.zeros((b, n_out), jnp.float32)
        init = (z, z, z, jnp.full((b, n), _INF, jnp.float32),
                jnp.zeros((b, 1), jnp.int32))
        qx, qy, qz, _, _ = lax.fori_loop(0, n_out, body, init)
        return qx, qy, qz

    px, py, pz = px_ref[...], py_ref[...], pz_ref[...]
    q1x, q1y, q1z = run(px, py, pz, _NQ1)
    q1x_ref[...] = q1x
    q1y_ref[...] = q1y
    q1z_ref[...] = q1z
    q2x, q2y, q2z = run(q1x, q1y, q1z, _NQ2)
    q2x_ref[...] = q2x
    q2y_ref[...] = q2y
    q2z_ref[...] = q2z


def _run_fps(px, py, pz):
    f32 = jnp.float32
    shapes = [jax.ShapeDtypeStruct((_B, _NQ1), f32)] * 3 + \
             [jax.ShapeDtypeStruct((_B, _NQ2), f32)] * 3
    return pl.pallas_call(_fps_body, out_shape=shapes)(px, py, pz)


def _select_body(qx_ref, qy_ref, qz_ref, px_ref, py_ref, pz_ref,
                 nbr_ref, msk_ref, score_ref, *, nq, n, rr):
    qx, qy, qz = qx_ref[0], qy_ref[0], qz_ref[0]
    px, py, pz = px_ref[0], py_ref[0], pz_ref[0]
    dx = qx - px
    dy = qy - py
    dz = qz - pz
    d2 = (dx * dx + dy * dy) + dz * dz
    within = d2 <= rr
    score_ref[...] = jnp.where(within, d2, _INF)
    cnt = jnp.sum(within.astype(jnp.int32), axis=1)
    rounds = jnp.minimum(_K, jnp.max(cnt))

    lane = lax.broadcasted_iota(jnp.int32, (nq, n), 1)
    kio = lax.broadcasted_iota(jnp.int32, (nq, _K), 1)

    def body(k, c):
        nbr, msk = c
        s = score_ref[...]
        m = jnp.min(s, axis=1, keepdims=True)
        idx = jnp.min(jnp.where(s == m, lane, n), axis=1, keepdims=True)
        valid = m < _INF
        hit = kio == k
        nbr = jnp.where(hit, idx, nbr)
        msk = jnp.where(jnp.logical_and(hit, valid), 1.0, msk)
        score_ref[...] = jnp.where(lane == idx, _INF, s)
        return nbr, msk

    nbr0 = jnp.zeros((nq, _K), jnp.int32)
    msk0 = jnp.zeros((nq, _K), jnp.float32)
    nbr, msk = lax.fori_loop(0, rounds, body, (nbr0, msk0))
    nbr_ref[0] = nbr
    msk_ref[0] = msk


def _run_select(qx, qy, qz, px, py, pz, nq, n, rr):
    qspec = pl.BlockSpec((1, nq, 1), lambda b: (b, 0, 0))
    pspec = pl.BlockSpec((1, 1, n), lambda b: (b, 0, 0))
    ospec = pl.BlockSpec((1, nq, _K), lambda b: (b, 0, 0))
    return pl.pallas_call(
        functools.partial(_select_body, nq=nq, n=n, rr=rr),
        grid=(_B,),
        in_specs=[qspec] * 3 + [pspec] * 3,
        out_specs=[ospec, ospec],
        out_shape=[jax.ShapeDtypeStruct((_B, nq, _K), jnp.int32),
                   jax.ShapeDtypeStruct((_B, nq, _K), jnp.float32)],
        scratch_shapes=[pltpu.VMEM((nq, n), jnp.float32)],
    )(qx.reshape(_B, nq, 1), qy.reshape(_B, nq, 1), qz.reshape(_B, nq, 1),
      px.reshape(_B, 1, n), py.reshape(_B, 1, n), pz.reshape(_B, 1, n))


_RB = 1024


def _gather_body(idx_ref, tab_ref, out_ref, *, n):
    idx = idx_ref[0]
    col = lax.broadcasted_iota(jnp.int32, (_RB, n), 1)
    oneh = (idx == col).astype(jnp.float32)
    out_ref[...] = jnp.dot(oneh, tab_ref[0],
                           preferred_element_type=jnp.float32,
                           precision=lax.Precision.HIGHEST)


def _run_gather(nbr, tab, nq, n, dp):
    nblk = nq * _K // _RB
    idx = nbr.reshape(_B * nblk, _RB, 1)
    return pl.pallas_call(
        functools.partial(_gather_body, n=n),
        grid=(_B, nblk),
        in_specs=[pl.BlockSpec((1, _RB, 1), lambda b, j: (b * nblk + j, 0, 0)),
                  pl.BlockSpec((1, n, dp), lambda b, j: (b, 0, 0))],
        out_specs=pl.BlockSpec((_RB, dp), lambda b, j: (b * nblk + j, 0)),
        out_shape=jax.ShapeDtypeStruct((_B * nq * _K, dp), jnp.float32),
    )(idx, tab)


_QB = 16


def _mlpmax_body(g_ref, pq_ref, mk_ref, w1_ref, a1_ref, w2_ref, a2_ref,
                 w3_ref, a3_ref, out_ref, *, dp):
    g = g_ref[...]
    g3 = g.reshape(_QB, _K, dp)
    feat = g3 - pq_ref[0][:, None, :]
    x = feat.reshape(_QB * _K, dp)
    for w_ref, a_ref in ((w1_ref, a1_ref), (w2_ref, a2_ref),
                         (w3_ref, a3_ref)):
        a = a_ref[...]
        x = jnp.maximum(
            jnp.dot(x, w_ref[...], preferred_element_type=jnp.float32)
            + a[0:1, :], 0.0)
        x = x * a[1:2, :] + a[2:3, :]
    cout = x.shape[1]
    h = x.reshape(_QB, _K, cout)
    h = jnp.where(mk_ref[0][:, :, None] > 0.0, h, -_INF)
    out_ref[0] = jnp.max(h, axis=1)


def _bn_aux(layer):
    s = layer['gamma'] / jnp.sqrt(layer['var'] + _EPS)
    t = layer['beta'] - layer['mean'] * s
    dout = s.shape[0]
    return jnp.concatenate(
        [layer['b'][None], s[None], t[None],
         jnp.zeros((5, dout), jnp.float32)], axis=0)


def _run_mlpmax(gathered, pq_pad, mask, layers, wpads, nq, dp):
    grid = (_B * nq // _QB,)
    cout = layers[2]['W'].shape[1]
    args = [gathered, pq_pad.reshape(_B * nq // _QB, _QB, dp),
            mask.reshape(_B * nq // _QB, _QB, _K)]
    in_specs = [pl.BlockSpec((_QB * _K, dp), lambda g: (g, 0)),
                pl.BlockSpec((1, _QB, dp), lambda g: (g, 0, 0)),
                pl.BlockSpec((1, _QB, _K), lambda g: (g, 0, 0))]
    for w, layer in zip(wpads, layers):
        a = _bn_aux(layer)
        args += [w, a]
        in_specs += [pl.BlockSpec(w.shape, lambda g: (0, 0)),
                     pl.BlockSpec(a.shape, lambda g: (0, 0))]
    out = pl.pallas_call(
        functools.partial(_mlpmax_body, dp=dp),
        grid=grid,
        in_specs=in_specs,
        out_specs=pl.BlockSpec((1, _QB, cout), lambda g: (g, 0, 0)),
        out_shape=jax.ShapeDtypeStruct((_B * nq // _QB, _QB, cout),
                                       jnp.float32),
    )(*args)
    return out.reshape(_B, nq, cout)


def _tf_body(x_ref, wqkv_ref, wo_ref, w1_ref, w2_ref,
             a3c_ref, ac_ref, af_ref, out_ref, *, c, h):
    hd = c // h
    scale = 1.0 / np.sqrt(hd)
    x = x_ref[0]
    qkv = jnp.dot(x, wqkv_ref[...],
                  preferred_element_type=jnp.float32,
                precision=lax.Precision.HIGHEST) + a3c_ref[0:1, :]
    q, k, v = qkv[:, :c], qkv[:, c:2 * c], qkv[:, 2 * c:]
    outs = []
    for i in range(h):
        sl = slice(i * hd, (i + 1) * hd)
        qh, kh, vh = q[:, sl], k[:, sl], v[:, sl]
        s = lax.dot_general(qh, kh, (((1,), (1,)), ((), ())),
                            preferred_element_type=jnp.float32,
                precision=lax.Precision.HIGHEST) * scale
        s = s - jnp.max(s, axis=1, keepdims=True)
        e = jnp.exp(s)
        att = e / jnp.sum(e, axis=1, keepdims=True)
        outs.append(jnp.dot(att, vh, preferred_element_type=jnp.float32,
                precision=lax.Precision.HIGHEST))
    o = jnp.concatenate(outs, axis=1)
    o = jnp.dot(o, wo_ref[...],
                preferred_element_type=jnp.float32,
                precision=lax.Precision.HIGHEST) + ac_ref[0:1, :]

    def ln(t, g, b):
        mu = jnp.mean(t, axis=1, keepdims=True)
        var = jnp.mean((t - mu) ** 2, axis=1, keepdims=True)
        return (t - mu) / jnp.sqrt(var + _EPS) * g + b

    x = ln(x + o, ac_ref[1:2, :], ac_ref[2:3, :])
    ff = jnp.maximum(
        jnp.dot(x, w1_ref[...], preferred_element_type=jnp.float32,
                precision=lax.Precision.HIGHEST)
        + af_ref[0:1, :], 0.0)
    ff = jnp.dot(ff, w2_ref[...],
                 preferred_element_type=jnp.float32,
                precision=lax.Precision.HIGHEST) + ac_ref[5:6, :]
    out_ref[0] = ln(x + ff, ac_ref[3:4, :], ac_ref[4:5, :])


def _run_transformer(p, x, heads):
    b, nq, c = x.shape
    ff = p['W1'].shape[1]
    z = jnp.zeros((1, c), jnp.float32)
    a3c = jnp.concatenate([p['bqkv'][None],
                           jnp.zeros((7, 3 * c), jnp.float32)], axis=0)
    ac = jnp.concatenate([p['bo'][None], p['ln1_g'][None], p['ln1_b'][None],
                          p['ln2_g'][None], p['ln2_b'][None], p['b2'][None],
                          z, z], axis=0)
    af = jnp.concatenate([p['b1'][None],
                          jnp.zeros((7, ff), jnp.float32)], axis=0)
    args = [x, p['Wqkv'], p['Wo'], p['W1'], p['W2'], a3c, ac, af]
    in_specs = [pl.BlockSpec((1, nq, c), lambda i: (i, 0, 0))]
    for a in args[1:]:
        in_specs.append(pl.BlockSpec(a.shape, lambda i: (0, 0)))
    return pl.pallas_call(
        functools.partial(_tf_body, c=c, h=heads),
        grid=(b,),
        in_specs=in_specs,
        out_specs=pl.BlockSpec((1, nq, c), lambda i: (i, 0, 0)),
        out_shape=jax.ShapeDtypeStruct((b, nq, c), jnp.float32),
    )(*args)


def _final_body(x_ref, w1_ref, a1_ref, w2_ref, a2_ref, w3_ref, a3_ref,
                lw1_ref, lb1_ref, lw2_ref, lb2_ref, lw3_ref, lb3_ref,
                out_ref):
    x = x_ref[0]
    for w_ref, a_ref in ((w1_ref, a1_ref), (w2_ref, a2_ref),
                         (w3_ref, a3_ref)):
        a = a_ref[...]
        x = jnp.maximum(
            jnp.dot(x, w_ref[...], preferred_element_type=jnp.float32,
                precision=lax.Precision.HIGHEST)
            + a[0:1, :], 0.0)
        x = x * a[1:2, :] + a[2:3, :]
    g = jnp.max(x, axis=0, keepdims=True)
    g = jnp.maximum(jnp.dot(g, lw1_ref[...],
                            preferred_element_type=jnp.float32,
                precision=lax.Precision.HIGHEST)
                    + lb1_ref[0:1, :], 0.0)
    g = jnp.maximum(jnp.dot(g, lw2_ref[...],
                            preferred_element_type=jnp.float32,
                precision=lax.Precision.HIGHEST)
                    + lb2_ref[0:1, :], 0.0)
    out_ref[0] = jnp.dot(g, lw3_ref[...],
                         preferred_element_type=jnp.float32,
                precision=lax.Precision.HIGHEST) + lb3_ref[0:1, :]


def _vec_aux(v):
    d = v.shape[0]
    return jnp.concatenate([v[None], jnp.zeros((7, d), jnp.float32)], axis=0)


def _run_final(feat, params):
    args = [feat]
    for layer in params['mlp3']:
        args += [layer['W'], _bn_aux(layer)]
    for nm in ('lin1', 'lin2', 'lin3'):
        args += [params[nm + '_W'], _vec_aux(params[nm + '_b'])]
    in_specs = [pl.BlockSpec((1, _NQ2, 259), lambda i: (i, 0, 0))]
    for a in args[1:]:
        in_specs.append(pl.BlockSpec(a.shape, lambda i: (0, 0)))
    out = pl.pallas_call(
        _final_body,
        grid=(_B,),
        in_specs=in_specs,
        out_specs=pl.BlockSpec((1, 1, 10), lambda i: (i, 0, 0)),
        out_shape=jax.ShapeDtypeStruct((_B, 1, 10), jnp.float32),
    )(*args)
    return out.reshape(_B, 10)


def _pad_cols(a, width):
    return jnp.concatenate(
        [a, jnp.zeros(a.shape[:-1] + (width - a.shape[-1],), jnp.float32)],
        axis=-1)


def kernel(data, params):
    if data.ndim == 3 and data.shape[1] == 3:
        data = jnp.transpose(data, (0, 2, 1))
    pos = data
    px, py, pz = pos[:, :, 0], pos[:, :, 1], pos[:, :, 2]

    q1x, q1y, q1z, q2x, q2y, q2z = _run_fps(px, py, pz)

    pos1 = jnp.stack([q1x, q1y, q1z], axis=-1)
    pq2 = jnp.stack([q2x, q2y, q2z], axis=-1)

    nbr1, msk1 = _run_select(q1x, q1y, q1z, px, py, pz,
                             _NQ1, _N, 0.2 * 0.2)
    mlp1 = params['mlp1']
    tab1 = _pad_cols(pos, 16)
    g1 = _run_gather(nbr1, tab1, _NQ1, _N, 16)
    pq1 = _pad_cols(pos1, 16)
    w1p = jnp.concatenate([mlp1[0]['W'],
                           jnp.zeros((13, 64), jnp.float32)], axis=0)
    x1 = _run_mlpmax(g1, pq1, msk1, mlp1,
                     [w1p, mlp1[1]['W'], mlp1[2]['W']], _NQ1, 16)
    x1 = _run_transformer(params['tf1'], x1, 4)

    nbr2, msk2 = _run_select(q2x, q2y, q2z, q1x, q1y, q1z,
                             _NQ2, _NQ1, 0.4 * 0.4)
    mlp2 = params['mlp2']
    tab2 = _pad_cols(jnp.concatenate([x1, pos1], axis=-1), 144)
    g2 = _run_gather(nbr2, tab2, _NQ2, _NQ1, 144)
    pq2p = jnp.concatenate(
        [jnp.zeros((_B, _NQ2, 128), jnp.float32), pq2,
         jnp.zeros((_B, _NQ2, 13), jnp.float32)], axis=-1)
    w2p = jnp.concatenate([mlp2[0]['W'],
                           jnp.zeros((13, 128), jnp.float32)], axis=0)
    x2 = _run_mlpmax(g2, pq2p, msk2, mlp2,
                     [w2p, mlp2[1]['W'], mlp2[2]['W']], _NQ2, 144)
    x2 = _run_transformer(params['tf2'], x2, 8)

    feat = jnp.concatenate([x2, pq2], axis=-1)
    return _run_final(feat, params)
